# baseline (speedup 1.0000x reference)
import functools

import jax
import jax.numpy as jnp
from jax import lax
from jax.experimental import pallas as pl
from jax.experimental.pallas import tpu as pltpu
from jax.experimental.pallas import tpu_sc as plsc

N = 10000
E = 320000
D = 128
W_COL = 144
NC = 2
NS = 16
NW = NC * NS
EPW = E // NW
CH = 40
NCH = EPW // CH
CPS = 10
SIDX = CPS * CH
NSTAGE = NCH // CPS
AN = 10240
ROWS_PER_TILE = AN // NS


def _lin_body(x_ref, wl_ref, bl_ref, wr_ref, br_ref, t_ref):
    xv = x_ref[...]
    t_ref[0] = jnp.dot(xv, wl_ref[...], preferred_element_type=jnp.float32) + bl_ref[...]
    t_ref[1] = jnp.dot(xv, wr_ref[...], preferred_element_type=jnp.float32) + br_ref[...]


def _linear_pair(x, Wl, bl, Wr, br):
    grid = (10,)
    blk = N // 10
    return pl.pallas_call(
        _lin_body,
        grid=grid,
        in_specs=[
            pl.BlockSpec((blk, D), lambda i: (i, 0)),
            pl.BlockSpec((D, D), lambda i: (0, 0)),
            pl.BlockSpec((1, D), lambda i: (0, 0)),
            pl.BlockSpec((D, D), lambda i: (0, 0)),
            pl.BlockSpec((1, D), lambda i: (0, 0)),
        ],
        out_specs=pl.BlockSpec((2, blk, D), lambda i: (0, i, 0)),
        out_shape=jax.ShapeDtypeStruct((2, N, D), jnp.float32),
    )(x, Wl, bl.reshape(1, D), Wr, br.reshape(1, D))


def _combine_lin_body(p_ref, bias_ref, wl_ref, bl_ref, wr_ref, br_ref, t_ref):
    ps = p_ref[0] + p_ref[1]
    feat = ps[:, :D]
    den = ps[:, D:D + 1]
    h = jnp.maximum(feat * (1.0 / (den + 1e-16)) + bias_ref[...], 0.0)
    t_ref[0] = jnp.dot(h, wl_ref[...], preferred_element_type=jnp.float32) + bl_ref[...]
    t_ref[1] = jnp.dot(h, wr_ref[...], preferred_element_type=jnp.float32) + br_ref[...]


def _combine_linear(p, bias, Wl, bl, Wr, br):
    grid = (10,)
    blk = N // 10
    return pl.pallas_call(
        _combine_lin_body,
        grid=grid,
        in_specs=[
            pl.BlockSpec((2, blk, W_COL), lambda i: (0, i, 0)),
            pl.BlockSpec((1, D), lambda i: (0, 0)),
            pl.BlockSpec((D, D), lambda i: (0, 0)),
            pl.BlockSpec((1, D), lambda i: (0, 0)),
            pl.BlockSpec((D, D), lambda i: (0, 0)),
            pl.BlockSpec((1, D), lambda i: (0, 0)),
        ],
        out_specs=pl.BlockSpec((2, blk, D), lambda i: (0, i, 0)),
        out_shape=jax.ShapeDtypeStruct((2, N, D), jnp.float32),
    )(p, bias.reshape(1, D), Wl, bl.reshape(1, D), Wr, br.reshape(1, D))


def _final_body(p_ref, bias_ref, out_ref):
    ps = p_ref[0] + p_ref[1]
    feat = ps[:, :D]
    den = ps[:, D:D + 1]
    out_ref[...] = feat * (1.0 / (den + 1e-16)) + bias_ref[...]


def _final_combine(p, bias):
    grid = (10,)
    blk = N // 10
    return pl.pallas_call(
        _final_body,
        grid=grid,
        in_specs=[
            pl.BlockSpec((2, blk, W_COL), lambda i: (0, i, 0)),
            pl.BlockSpec((1, D), lambda i: (0, 0)),
        ],
        out_specs=pl.BlockSpec((blk, D), lambda i: (i, 0)),
        out_shape=jax.ShapeDtypeStruct((N, D), jnp.float32),
    )(p, bias.reshape(1, D))


def _edge_body(t_hbm, cidx_hbm, dst_hbm, att_hbm, p_hbm,
               cidx_i, dst_i, ab_buf, out_buf, att_v,
               accum, sa0, sa1, sc0, sc1, si_sem):
    c = lax.axis_index("c")
    s = lax.axis_index("s")
    w = s * NC + c
    base = w * EPW
    cbase = 2 * base

    pltpu.sync_copy(cidx_hbm.at[pl.ds(cbase, 2 * SIDX)], cidx_i.at[0])
    pltpu.sync_copy(dst_hbm.at[pl.ds(base, SIDX)], dst_i.at[0])
    pltpu.sync_copy(att_hbm, att_v)
    att_ks = [att_v[pl.ds(k * 16, 16)] for k in range(8)]
    lane = lax.iota(jnp.int32, 16)
    zero16 = jnp.zeros((16,), jnp.float32)

    def _z(r, carry):
        for k in range(W_COL // 16):
            out_buf[r, pl.ds(k * 16, 16)] = zero16
        return carry
    lax.fori_loop(0, 2 * CH, _z, 0)
    for i in range(ROWS_PER_TILE // (2 * CH)):
        pltpu.sync_copy(out_buf.at[pl.ds(0, 2 * CH)],
                        accum.at[pl.ds(s * ROWS_PER_TILE + i * 2 * CH, 2 * CH)])
    plsc.subcore_barrier()

    sems_a = (sa0, sa1)
    sems_c = (sc0, sc1)

    def maybe_prefetch(i):
        si = lax.div(i, CPS)
        cond = (lax.rem(i, CPS) == 2) & (si + 1 < NSTAGE)

        def do():
            p = lax.rem(si + 1, 2)
            pltpu.async_copy(cidx_hbm.at[pl.ds(cbase + (si + 1) * 2 * SIDX,
                                               2 * SIDX)],
                             cidx_i.at[p], si_sem)
            pltpu.async_copy(dst_hbm.at[pl.ds(base + (si + 1) * SIDX, SIDX)],
                             dst_i.at[p], si_sem)
        pl.when(cond)(do)

    def maybe_wait_idx(i):
        si = lax.div(i, CPS)
        cond = (lax.rem(i, CPS) == CPS - 3) & (si + 1 < NSTAGE)

        def do():
            pltpu.make_async_copy(cidx_hbm.at[pl.ds(cbase, 2 * SIDX)],
                                  cidx_i.at[0], si_sem).wait()
            pltpu.make_async_copy(dst_hbm.at[pl.ds(base, SIDX)],
                                  dst_i.at[0], si_sem).wait()
        pl.when(cond)(do)

    def issue(i, slot):
        p = lax.rem(lax.div(i, CPS), 2)
        o = lax.rem(i, CPS) * 2 * CH
        pltpu.async_copy(t_hbm.at[cidx_i.at[p, pl.ds(o, 2 * CH)]],
                         ab_buf.at[pl.ds(slot * 2 * CH, 2 * CH)], sems_a[slot])

    def wait(slot):
        pltpu.make_async_copy(t_hbm.at[cidx_i.at[0, pl.ds(0, 2 * CH)]],
                              ab_buf.at[pl.ds(slot * 2 * CH, 2 * CH)],
                              sems_a[slot]).wait()

    def wait_scatter(slot):
        pltpu.make_async_copy(out_buf.at[pl.ds(slot * CH, CH)],
                              accum.at[dst_i.at[0, pl.ds(0, CH)]],
                              sems_c[slot]).wait()

    def compute_scatter(i, slot):
        rb = slot * CH
        ra = slot * 2 * CH

        @plsc.parallel_loop(0, CH, 1, unroll=4)
        def _edges(j):
            r = rb + j
            acc = zero16
            avs = []
            for k in range(8):
                a = ab_buf[ra + j, pl.ds(k * 16, 16)]
                avs.append(a)
                b = ab_buf[ra + CH + j, pl.ds(k * 16, 16)]
                z = a + b
                lrelu = jnp.maximum(z, z * 0.2)
                acc = acc + lrelu * att_ks[k]
            parts = [acc[l] for l in range(16)]
            while len(parts) > 1:
                parts = [parts[m] + parts[m + 1]
                         for m in range(0, len(parts), 2)]
            ex_vec = jnp.exp(parts[0] + zero16)
            for k in range(8):
                out_buf[r, pl.ds(k * 16, 16)] = avs[k] * ex_vec
            out_buf[r, pl.ds(D, 16)] = jnp.where(lane == 0, ex_vec, 0.0)

        p = lax.rem(lax.div(i, CPS), 2)
        o = lax.rem(i, CPS) * CH
        pltpu.async_copy(out_buf.at[pl.ds(rb, CH)],
                         accum.at[dst_i.at[p, pl.ds(o, CH)]],
                         sems_c[slot], add=True)

    issue(0, 0)

    def two_chunks(t, carry):
        i0 = t * 2
        maybe_prefetch(i0)
        maybe_wait_idx(i0)
        issue(i0 + 1, 1)
        wait(0)
        pl.when(t > 0)(lambda: wait_scatter(0))
        compute_scatter(i0, 0)
        maybe_prefetch(i0 + 1)
        maybe_wait_idx(i0 + 1)
        pl.when(i0 + 2 < NCH)(lambda: issue(i0 + 2, 0))
        wait(1)
        pl.when(t > 0)(lambda: wait_scatter(1))
        compute_scatter(i0 + 1, 1)
        return carry

    lax.fori_loop(0, NCH // 2, two_chunks, 0)
    wait_scatter(0)
    wait_scatter(1)

    plsc.subcore_barrier()
    pltpu.sync_copy(accum.at[pl.ds(s * ROWS_PER_TILE, ROWS_PER_TILE)],
                    p_hbm.at[c, pl.ds(s * ROWS_PER_TILE, ROWS_PER_TILE)])


def _edge_pass(t, cidx, dst, att):
    mesh = plsc.VectorSubcoreMesh(core_axis_name="c", subcore_axis_name="s")
    return pl.kernel(
        _edge_body,
        out_type=jax.ShapeDtypeStruct((NC, AN, W_COL), jnp.float32),
        mesh=mesh,
        compiler_params=pltpu.CompilerParams(use_tc_tiling_on_sc=False),
        scratch_types=[
            pltpu.VMEM((2, 2 * SIDX), jnp.int32),
            pltpu.VMEM((2, SIDX), jnp.int32),
            pltpu.VMEM((4 * CH, D), jnp.float32),
            pltpu.VMEM((2 * CH, W_COL), jnp.float32),
            pltpu.VMEM((D,), jnp.float32),
            pltpu.VMEM_SHARED((AN, W_COL), jnp.float32),
            pltpu.SemaphoreType.DMA,
            pltpu.SemaphoreType.DMA,
            pltpu.SemaphoreType.DMA,
            pltpu.SemaphoreType.DMA,
            pltpu.SemaphoreType.DMA,
        ],
    )(t, cidx, dst, att)


def kernel(x, edge_idx, W1_l, b1_l, W1_r, b1_r, att1, bias1,
           W2_l, b2_l, W2_r, b2_r, att2, bias2):
    eidx = edge_idx.astype(jnp.int32)
    src = eidx[0]
    dst = eidx[1]
    cidx = jnp.concatenate(
        [src.reshape(-1, CH), dst.reshape(-1, CH) + N], axis=1).reshape(-1)
    t1 = _linear_pair(x, W1_l, b1_l, W1_r, b1_r)
    p1 = _edge_pass(t1.reshape(2 * N, D), cidx, dst, att1)
    t2 = _combine_linear(p1, bias1, W2_l, b2_l, W2_r, b2_r)
    p2 = _edge_pass(t2.reshape(2 * N, D), cidx, dst, att2)
    return _final_combine(p2, bias2)


# speedup vs baseline: 1.0405x; 1.0405x over previous
import functools

import jax
import jax.numpy as jnp
from jax import lax
from jax.experimental import pallas as pl
from jax.experimental.pallas import tpu as pltpu
from jax.experimental.pallas import tpu_sc as plsc

N = 10000
E = 320000
D = 128
W_COL = 144
NC = 2
NS = 16
NW = NC * NS
EPW = E // NW
CH = 40
NCH = EPW // CH
CPS = 25
SIDX = CPS * CH
NSTAGE = NCH // CPS
AN = 10240
ROWS_PER_TILE = AN // NS


def _lin_body(x_ref, wl_ref, bl_ref, wr_ref, br_ref, t_ref):
    xv = x_ref[...]
    t_ref[0] = jnp.dot(xv, wl_ref[...], preferred_element_type=jnp.float32) + bl_ref[...]
    t_ref[1] = jnp.dot(xv, wr_ref[...], preferred_element_type=jnp.float32) + br_ref[...]


def _linear_pair(x, Wl, bl, Wr, br):
    grid = (10,)
    blk = N // 10
    return pl.pallas_call(
        _lin_body,
        grid=grid,
        in_specs=[
            pl.BlockSpec((blk, D), lambda i: (i, 0)),
            pl.BlockSpec((D, D), lambda i: (0, 0)),
            pl.BlockSpec((1, D), lambda i: (0, 0)),
            pl.BlockSpec((D, D), lambda i: (0, 0)),
            pl.BlockSpec((1, D), lambda i: (0, 0)),
        ],
        out_specs=pl.BlockSpec((2, blk, D), lambda i: (0, i, 0)),
        out_shape=jax.ShapeDtypeStruct((2, N, D), jnp.float32),
    )(x, Wl, bl.reshape(1, D), Wr, br.reshape(1, D))


def _combine_lin_body(p_ref, bias_ref, wl_ref, bl_ref, wr_ref, br_ref, t_ref):
    ps = p_ref[0] + p_ref[1]
    feat = ps[:, :D]
    den = ps[:, D:D + 1]
    h = jnp.maximum(feat * (1.0 / (den + 1e-16)) + bias_ref[...], 0.0)
    t_ref[0] = jnp.dot(h, wl_ref[...], preferred_element_type=jnp.float32) + bl_ref[...]
    t_ref[1] = jnp.dot(h, wr_ref[...], preferred_element_type=jnp.float32) + br_ref[...]


def _combine_linear(p, bias, Wl, bl, Wr, br):
    grid = (10,)
    blk = N // 10
    return pl.pallas_call(
        _combine_lin_body,
        grid=grid,
        in_specs=[
            pl.BlockSpec((2, blk, W_COL), lambda i: (0, i, 0)),
            pl.BlockSpec((1, D), lambda i: (0, 0)),
            pl.BlockSpec((D, D), lambda i: (0, 0)),
            pl.BlockSpec((1, D), lambda i: (0, 0)),
            pl.BlockSpec((D, D), lambda i: (0, 0)),
            pl.BlockSpec((1, D), lambda i: (0, 0)),
        ],
        out_specs=pl.BlockSpec((2, blk, D), lambda i: (0, i, 0)),
        out_shape=jax.ShapeDtypeStruct((2, N, D), jnp.float32),
    )(p, bias.reshape(1, D), Wl, bl.reshape(1, D), Wr, br.reshape(1, D))


def _final_body(p_ref, bias_ref, out_ref):
    ps = p_ref[0] + p_ref[1]
    feat = ps[:, :D]
    den = ps[:, D:D + 1]
    out_ref[...] = feat * (1.0 / (den + 1e-16)) + bias_ref[...]


def _final_combine(p, bias):
    grid = (10,)
    blk = N // 10
    return pl.pallas_call(
        _final_body,
        grid=grid,
        in_specs=[
            pl.BlockSpec((2, blk, W_COL), lambda i: (0, i, 0)),
            pl.BlockSpec((1, D), lambda i: (0, 0)),
        ],
        out_specs=pl.BlockSpec((blk, D), lambda i: (i, 0)),
        out_shape=jax.ShapeDtypeStruct((N, D), jnp.float32),
    )(p, bias.reshape(1, D))


def _edge_body(xl_hbm, xr_hbm, src_hbm, dst_hbm, att_hbm, p_hbm,
               src_i, dst_i, a_buf, b_buf, out_buf, att_v,
               accum, sa0, sa1, sb0, sb1, sc0, sc1, si_sem):
    c = lax.axis_index("c")
    s = lax.axis_index("s")
    w = s * NC + c
    base = w * EPW

    pltpu.sync_copy(src_hbm.at[pl.ds(base, SIDX)], src_i.at[0])
    pltpu.sync_copy(dst_hbm.at[pl.ds(base, SIDX)], dst_i.at[0])
    pltpu.sync_copy(att_hbm, att_v)
    att_ks = [att_v[pl.ds(k * 16, 16)] for k in range(8)]
    lane = lax.iota(jnp.int32, 16)
    zero16 = jnp.zeros((16,), jnp.float32)

    def _z(r, carry):
        for k in range(W_COL // 16):
            out_buf[r, pl.ds(k * 16, 16)] = zero16
        return carry
    lax.fori_loop(0, 2 * CH, _z, 0)
    for i in range(ROWS_PER_TILE // (2 * CH)):
        pltpu.sync_copy(out_buf.at[pl.ds(0, 2 * CH)],
                        accum.at[pl.ds(s * ROWS_PER_TILE + i * 2 * CH, 2 * CH)])
    plsc.subcore_barrier()

    sems_a = (sa0, sa1)
    sems_b = (sb0, sb1)
    sems_c = (sc0, sc1)

    def maybe_prefetch(i):
        si = lax.div(i, CPS)
        cond = (lax.rem(i, CPS) == 2) & (si + 1 < NSTAGE)

        def do():
            p = lax.rem(si + 1, 2)
            off = base + (si + 1) * SIDX
            pltpu.async_copy(src_hbm.at[pl.ds(off, SIDX)], src_i.at[p], si_sem)
            pltpu.async_copy(dst_hbm.at[pl.ds(off, SIDX)], dst_i.at[p], si_sem)
        pl.when(cond)(do)

    def maybe_wait_idx(i):
        si = lax.div(i, CPS)
        cond = (lax.rem(i, CPS) == CPS - 3) & (si + 1 < NSTAGE)

        def do():
            pltpu.make_async_copy(src_hbm.at[pl.ds(base, SIDX)],
                                  src_i.at[0], si_sem).wait()
            pltpu.make_async_copy(dst_hbm.at[pl.ds(base, SIDX)],
                                  dst_i.at[0], si_sem).wait()
        pl.when(cond)(do)

    def idx_slice(ref, i):
        p = lax.rem(lax.div(i, CPS), 2)
        o = lax.rem(i, CPS) * CH
        return ref.at[p, pl.ds(o, CH)]

    def issue(i, slot):
        pltpu.async_copy(xl_hbm.at[idx_slice(src_i, i)],
                         a_buf.at[pl.ds(slot * CH, CH)], sems_a[slot])
        pltpu.async_copy(xr_hbm.at[idx_slice(dst_i, i)],
                         b_buf.at[pl.ds(slot * CH, CH)], sems_b[slot])

    def wait(slot):
        pltpu.make_async_copy(xl_hbm.at[src_i.at[0, pl.ds(0, CH)]],
                              a_buf.at[pl.ds(slot * CH, CH)], sems_a[slot]).wait()
        pltpu.make_async_copy(xr_hbm.at[src_i.at[0, pl.ds(0, CH)]],
                              b_buf.at[pl.ds(slot * CH, CH)], sems_b[slot]).wait()

    def wait_scatter(slot):
        pltpu.make_async_copy(out_buf.at[pl.ds(slot * CH, CH)],
                              accum.at[src_i.at[0, pl.ds(0, CH)]],
                              sems_c[slot]).wait()

    def compute_scatter(i, slot):
        rb = slot * CH

        @plsc.parallel_loop(0, CH, 1, unroll=4)
        def _edges(j):
            r = rb + j
            acc = zero16
            avs = []
            for k in range(8):
                a = a_buf[r, pl.ds(k * 16, 16)]
                avs.append(a)
                b = b_buf[r, pl.ds(k * 16, 16)]
                z = a + b
                lrelu = jnp.maximum(z, z * 0.2)
                acc = acc + lrelu * att_ks[k]
            parts = [acc[l] for l in range(16)]
            while len(parts) > 1:
                parts = [parts[m] + parts[m + 1]
                         for m in range(0, len(parts), 2)]
            ex_vec = jnp.exp(parts[0] + zero16)
            for k in range(8):
                out_buf[r, pl.ds(k * 16, 16)] = avs[k] * ex_vec
            out_buf[r, pl.ds(D, 16)] = jnp.where(lane == 0, ex_vec, 0.0)

        pltpu.async_copy(out_buf.at[pl.ds(rb, CH)],
                         accum.at[idx_slice(dst_i, i)],
                         sems_c[slot], add=True)

    issue(0, 0)

    def two_chunks(t, carry):
        i0 = t * 2
        maybe_prefetch(i0)
        maybe_wait_idx(i0)
        issue(i0 + 1, 1)
        wait(0)
        pl.when(t > 0)(lambda: wait_scatter(0))
        compute_scatter(i0, 0)
        maybe_prefetch(i0 + 1)
        maybe_wait_idx(i0 + 1)
        pl.when(i0 + 2 < NCH)(lambda: issue(i0 + 2, 0))
        wait(1)
        pl.when(t > 0)(lambda: wait_scatter(1))
        compute_scatter(i0 + 1, 1)
        return carry

    lax.fori_loop(0, NCH // 2, two_chunks, 0)
    wait_scatter(0)
    wait_scatter(1)

    plsc.subcore_barrier()
    pltpu.sync_copy(accum.at[pl.ds(s * ROWS_PER_TILE, ROWS_PER_TILE)],
                    p_hbm.at[c, pl.ds(s * ROWS_PER_TILE, ROWS_PER_TILE)])


def _edge_pass(xl, xr, src, dst, att):
    mesh = plsc.VectorSubcoreMesh(core_axis_name="c", subcore_axis_name="s")
    return pl.kernel(
        _edge_body,
        out_type=jax.ShapeDtypeStruct((NC, AN, W_COL), jnp.float32),
        mesh=mesh,
        compiler_params=pltpu.CompilerParams(use_tc_tiling_on_sc=False),
        scratch_types=[
            pltpu.VMEM((2, SIDX), jnp.int32),
            pltpu.VMEM((2, SIDX), jnp.int32),
            pltpu.VMEM((2 * CH, D), jnp.float32),
            pltpu.VMEM((2 * CH, D), jnp.float32),
            pltpu.VMEM((2 * CH, W_COL), jnp.float32),
            pltpu.VMEM((D,), jnp.float32),
            pltpu.VMEM_SHARED((AN, W_COL), jnp.float32),
            pltpu.SemaphoreType.DMA,
            pltpu.SemaphoreType.DMA,
            pltpu.SemaphoreType.DMA,
            pltpu.SemaphoreType.DMA,
            pltpu.SemaphoreType.DMA,
            pltpu.SemaphoreType.DMA,
            pltpu.SemaphoreType.DMA,
        ],
    )(xl, xr, src, dst, att)


def kernel(x, edge_idx, W1_l, b1_l, W1_r, b1_r, att1, bias1,
           W2_l, b2_l, W2_r, b2_r, att2, bias2):
    eidx = edge_idx.astype(jnp.int32)
    src = eidx[0]
    dst = eidx[1]
    t1 = _linear_pair(x, W1_l, b1_l, W1_r, b1_r)
    p1 = _edge_pass(t1[0], t1[1], src, dst, att1)
    t2 = _combine_linear(p1, bias1, W2_l, b2_l, W2_r, b2_r)
    p2 = _edge_pass(t2[0], t2[1], src, dst, att2)
    return _final_combine(p2, bias2)


# speedup vs baseline: 1.0645x; 1.0230x over previous
import functools

import jax
import jax.numpy as jnp
from jax import lax
from jax.experimental import pallas as pl
from jax.experimental.pallas import tpu as pltpu
from jax.experimental.pallas import tpu_sc as plsc

N = 10000
E = 320000
D = 128
W_COL = 144
NC = 2
NS = 16
NW = NC * NS
EPW = E // NW
CH = 40
NCH = EPW // CH
CPS = 25
SIDX = CPS * CH
NSTAGE = NCH // CPS
AN = 10240
ROWS_PER_TILE = AN // NS


def _lin_body(x_ref, wl_ref, bl_ref, wr_ref, br_ref, xl_ref, xr_ref):
    xv = x_ref[...]
    xl_ref[...] = jnp.dot(xv, wl_ref[...], preferred_element_type=jnp.float32) + bl_ref[...]
    xr_ref[...] = jnp.dot(xv, wr_ref[...], preferred_element_type=jnp.float32) + br_ref[...]


def _linear_pair(x, Wl, bl, Wr, br):
    grid = (10,)
    blk = N // 10
    return pl.pallas_call(
        _lin_body,
        grid=grid,
        in_specs=[
            pl.BlockSpec((blk, D), lambda i: (i, 0)),
            pl.BlockSpec((D, D), lambda i: (0, 0)),
            pl.BlockSpec((1, D), lambda i: (0, 0)),
            pl.BlockSpec((D, D), lambda i: (0, 0)),
            pl.BlockSpec((1, D), lambda i: (0, 0)),
        ],
        out_specs=[
            pl.BlockSpec((blk, D), lambda i: (i, 0)),
            pl.BlockSpec((blk, D), lambda i: (i, 0)),
        ],
        out_shape=[
            jax.ShapeDtypeStruct((N, D), jnp.float32),
            jax.ShapeDtypeStruct((N, D), jnp.float32),
        ],
    )(x, Wl, bl.reshape(1, D), Wr, br.reshape(1, D))


def _combine_lin_body(p_ref, bias_ref, wl_ref, bl_ref, wr_ref, br_ref, xl_ref, xr_ref):
    ps = p_ref[0] + p_ref[1]
    feat = ps[:, :D]
    den = ps[:, D:D + 1]
    h = jnp.maximum(feat * (1.0 / (den + 1e-16)) + bias_ref[...], 0.0)
    xl_ref[...] = jnp.dot(h, wl_ref[...], preferred_element_type=jnp.float32) + bl_ref[...]
    xr_ref[...] = jnp.dot(h, wr_ref[...], preferred_element_type=jnp.float32) + br_ref[...]


def _combine_linear(p, bias, Wl, bl, Wr, br):
    grid = (10,)
    blk = N // 10
    return pl.pallas_call(
        _combine_lin_body,
        grid=grid,
        in_specs=[
            pl.BlockSpec((2, blk, W_COL), lambda i: (0, i, 0)),
            pl.BlockSpec((1, D), lambda i: (0, 0)),
            pl.BlockSpec((D, D), lambda i: (0, 0)),
            pl.BlockSpec((1, D), lambda i: (0, 0)),
            pl.BlockSpec((D, D), lambda i: (0, 0)),
            pl.BlockSpec((1, D), lambda i: (0, 0)),
        ],
        out_specs=[
            pl.BlockSpec((blk, D), lambda i: (i, 0)),
            pl.BlockSpec((blk, D), lambda i: (i, 0)),
        ],
        out_shape=[
            jax.ShapeDtypeStruct((N, D), jnp.float32),
            jax.ShapeDtypeStruct((N, D), jnp.float32),
        ],
    )(p, bias.reshape(1, D), Wl, bl.reshape(1, D), Wr, br.reshape(1, D))


def _final_body(p_ref, bias_ref, out_ref):
    ps = p_ref[0] + p_ref[1]
    feat = ps[:, :D]
    den = ps[:, D:D + 1]
    out_ref[...] = feat * (1.0 / (den + 1e-16)) + bias_ref[...]


def _final_combine(p, bias):
    grid = (10,)
    blk = N // 10
    return pl.pallas_call(
        _final_body,
        grid=grid,
        in_specs=[
            pl.BlockSpec((2, blk, W_COL), lambda i: (0, i, 0)),
            pl.BlockSpec((1, D), lambda i: (0, 0)),
        ],
        out_specs=pl.BlockSpec((blk, D), lambda i: (i, 0)),
        out_shape=jax.ShapeDtypeStruct((N, D), jnp.float32),
    )(p, bias.reshape(1, D))


def _edge_body(xl_hbm, xr_hbm, src_hbm, dst_hbm, att_hbm, p_hbm,
               src_i, dst_i, a_buf, b_buf, out_buf, att_v,
               accum, sa0, sa1, sb0, sb1, sc0, sc1, si_sem):
    c = lax.axis_index("c")
    s = lax.axis_index("s")
    w = s * NC + c
    base = w * EPW

    pltpu.sync_copy(src_hbm.at[pl.ds(base, SIDX)], src_i.at[0])
    pltpu.sync_copy(dst_hbm.at[pl.ds(base, SIDX)], dst_i.at[0])
    pltpu.sync_copy(att_hbm, att_v)
    att_ks = [att_v[pl.ds(k * 16, 16)] for k in range(8)]
    lane = lax.iota(jnp.int32, 16)
    zero16 = jnp.zeros((16,), jnp.float32)

    def _z(r, carry):
        for k in range(W_COL // 16):
            out_buf[r, pl.ds(k * 16, 16)] = zero16
        return carry
    lax.fori_loop(0, 2 * CH, _z, 0)
    for i in range(ROWS_PER_TILE // (2 * CH)):
        pltpu.sync_copy(out_buf.at[pl.ds(0, 2 * CH)],
                        accum.at[pl.ds(s * ROWS_PER_TILE + i * 2 * CH, 2 * CH)])
    plsc.subcore_barrier()

    sems_a = (sa0, sa1)
    sems_b = (sb0, sb1)
    sems_c = (sc0, sc1)

    def maybe_prefetch(i):
        si = lax.div(i, CPS)
        cond = (lax.rem(i, CPS) == 2) & (si + 1 < NSTAGE)

        def do():
            p = lax.rem(si + 1, 2)
            off = base + (si + 1) * SIDX
            pltpu.async_copy(src_hbm.at[pl.ds(off, SIDX)], src_i.at[p], si_sem)
            pltpu.async_copy(dst_hbm.at[pl.ds(off, SIDX)], dst_i.at[p], si_sem)
        pl.when(cond)(do)

    def maybe_wait_idx(i):
        si = lax.div(i, CPS)
        cond = (lax.rem(i, CPS) == CPS - 3) & (si + 1 < NSTAGE)

        def do():
            pltpu.make_async_copy(src_hbm.at[pl.ds(base, SIDX)],
                                  src_i.at[0], si_sem).wait()
            pltpu.make_async_copy(dst_hbm.at[pl.ds(base, SIDX)],
                                  dst_i.at[0], si_sem).wait()
        pl.when(cond)(do)

    def idx_slice(ref, i):
        p = lax.rem(lax.div(i, CPS), 2)
        o = lax.rem(i, CPS) * CH
        return ref.at[p, pl.ds(o, CH)]

    def issue(i, slot):
        pltpu.async_copy(xl_hbm.at[idx_slice(src_i, i)],
                         a_buf.at[pl.ds(slot * CH, CH)], sems_a[slot])
        pltpu.async_copy(xr_hbm.at[idx_slice(dst_i, i)],
                         b_buf.at[pl.ds(slot * CH, CH)], sems_b[slot])

    def wait(slot):
        pltpu.make_async_copy(xl_hbm.at[src_i.at[0, pl.ds(0, CH)]],
                              a_buf.at[pl.ds(slot * CH, CH)], sems_a[slot]).wait()
        pltpu.make_async_copy(xr_hbm.at[src_i.at[0, pl.ds(0, CH)]],
                              b_buf.at[pl.ds(slot * CH, CH)], sems_b[slot]).wait()

    def wait_scatter(slot):
        pltpu.make_async_copy(out_buf.at[pl.ds(slot * CH, CH)],
                              accum.at[src_i.at[0, pl.ds(0, CH)]],
                              sems_c[slot]).wait()

    def compute_scatter(i, slot):
        rb = slot * CH

        @plsc.parallel_loop(0, CH, 1, unroll=4)
        def _edges(j):
            r = rb + j
            acc = zero16
            avs = []
            for k in range(8):
                a = a_buf[r, pl.ds(k * 16, 16)]
                avs.append(a)
                b = b_buf[r, pl.ds(k * 16, 16)]
                z = a + b
                lrelu = jnp.maximum(z, z * 0.2)
                acc = acc + lrelu * att_ks[k]
            parts = [acc[l] for l in range(16)]
            while len(parts) > 1:
                parts = [parts[m] + parts[m + 1]
                         for m in range(0, len(parts), 2)]
            ex_vec = jnp.exp(parts[0] + zero16)
            for k in range(8):
                out_buf[r, pl.ds(k * 16, 16)] = avs[k] * ex_vec
            out_buf[r, pl.ds(D, 16)] = jnp.where(lane == 0, ex_vec, 0.0)

        pltpu.async_copy(out_buf.at[pl.ds(rb, CH)],
                         accum.at[idx_slice(dst_i, i)],
                         sems_c[slot], add=True)

    issue(0, 0)

    def two_chunks(t, carry):
        i0 = t * 2
        maybe_prefetch(i0)
        maybe_wait_idx(i0)
        issue(i0 + 1, 1)
        wait(0)
        pl.when(t > 0)(lambda: wait_scatter(0))
        compute_scatter(i0, 0)
        maybe_prefetch(i0 + 1)
        maybe_wait_idx(i0 + 1)
        pl.when(i0 + 2 < NCH)(lambda: issue(i0 + 2, 0))
        wait(1)
        pl.when(t > 0)(lambda: wait_scatter(1))
        compute_scatter(i0 + 1, 1)
        return carry

    lax.fori_loop(0, NCH // 2, two_chunks, 0)
    wait_scatter(0)
    wait_scatter(1)

    plsc.subcore_barrier()
    pltpu.sync_copy(accum.at[pl.ds(s * ROWS_PER_TILE, ROWS_PER_TILE)],
                    p_hbm.at[c, pl.ds(s * ROWS_PER_TILE, ROWS_PER_TILE)])


def _edge_pass(xl, xr, src, dst, att):
    mesh = plsc.VectorSubcoreMesh(core_axis_name="c", subcore_axis_name="s")
    return pl.kernel(
        _edge_body,
        out_type=jax.ShapeDtypeStruct((NC, AN, W_COL), jnp.float32),
        mesh=mesh,
        compiler_params=pltpu.CompilerParams(use_tc_tiling_on_sc=False),
        scratch_types=[
            pltpu.VMEM((2, SIDX), jnp.int32),
            pltpu.VMEM((2, SIDX), jnp.int32),
            pltpu.VMEM((2 * CH, D), jnp.float32),
            pltpu.VMEM((2 * CH, D), jnp.float32),
            pltpu.VMEM((2 * CH, W_COL), jnp.float32),
            pltpu.VMEM((D,), jnp.float32),
            pltpu.VMEM_SHARED((AN, W_COL), jnp.float32),
            pltpu.SemaphoreType.DMA,
            pltpu.SemaphoreType.DMA,
            pltpu.SemaphoreType.DMA,
            pltpu.SemaphoreType.DMA,
            pltpu.SemaphoreType.DMA,
            pltpu.SemaphoreType.DMA,
            pltpu.SemaphoreType.DMA,
        ],
    )(xl, xr, src, dst, att)


def kernel(x, edge_idx, W1_l, b1_l, W1_r, b1_r, att1, bias1,
           W2_l, b2_l, W2_r, b2_r, att2, bias2):
    eidx = edge_idx.astype(jnp.int32)
    src = eidx[0]
    dst = eidx[1]
    xl1, xr1 = _linear_pair(x, W1_l, b1_l, W1_r, b1_r)
    p1 = _edge_pass(xl1, xr1, src, dst, att1)
    xl2, xr2 = _combine_linear(p1, bias1, W2_l, b2_l, W2_r, b2_r)
    p2 = _edge_pass(xl2, xr2, src, dst, att2)
    return _final_combine(p2, bias2)


# speedup vs baseline: 1.1033x; 1.0365x over previous
import functools

import jax
import jax.numpy as jnp
from jax import lax
from jax.experimental import pallas as pl
from jax.experimental.pallas import tpu as pltpu
from jax.experimental.pallas import tpu_sc as plsc

N = 10000
E = 320000
D = 128
W_COL = 144
NC = 2
NS = 16
NW = NC * NS
EPW = E // NW
CH = 40
NCH = EPW // CH
CPS = 25
SIDX = CPS * CH
NSTAGE = NCH // CPS
AN = 10240
ROWS_PER_TILE = AN // NS


def _lin_body(x_ref, wl_ref, bl_ref, wr_ref, br_ref, xl_ref, xr_ref):
    xv = x_ref[...]
    xl_ref[...] = jnp.dot(xv, wl_ref[...], preferred_element_type=jnp.float32) + bl_ref[...]
    xr_ref[...] = jnp.dot(xv, wr_ref[...], preferred_element_type=jnp.float32) + br_ref[...]


def _linear_pair(x, Wl, bl, Wr, br):
    grid = (10,)
    blk = N // 10
    return pl.pallas_call(
        _lin_body,
        grid=grid,
        in_specs=[
            pl.BlockSpec((blk, D), lambda i: (i, 0)),
            pl.BlockSpec((D, D), lambda i: (0, 0)),
            pl.BlockSpec((1, D), lambda i: (0, 0)),
            pl.BlockSpec((D, D), lambda i: (0, 0)),
            pl.BlockSpec((1, D), lambda i: (0, 0)),
        ],
        out_specs=[
            pl.BlockSpec((blk, D), lambda i: (i, 0)),
            pl.BlockSpec((blk, D), lambda i: (i, 0)),
        ],
        out_shape=[
            jax.ShapeDtypeStruct((N, D), jnp.float32),
            jax.ShapeDtypeStruct((N, D), jnp.float32),
        ],
    )(x, Wl, bl.reshape(1, D), Wr, br.reshape(1, D))


def _combine_lin_body(p_ref, bias_ref, wl_ref, bl_ref, wr_ref, br_ref, xl_ref, xr_ref):
    ps = p_ref[0] + p_ref[1]
    feat = ps[:, :D]
    den = ps[:, D:D + 1]
    h = jnp.maximum(feat * (1.0 / (den + 1e-16)) + bias_ref[...], 0.0)
    xl_ref[...] = jnp.dot(h, wl_ref[...], preferred_element_type=jnp.float32) + bl_ref[...]
    xr_ref[...] = jnp.dot(h, wr_ref[...], preferred_element_type=jnp.float32) + br_ref[...]


def _combine_linear(p, bias, Wl, bl, Wr, br):
    grid = (10,)
    blk = N // 10
    return pl.pallas_call(
        _combine_lin_body,
        grid=grid,
        in_specs=[
            pl.BlockSpec((2, blk, W_COL), lambda i: (0, i, 0)),
            pl.BlockSpec((1, D), lambda i: (0, 0)),
            pl.BlockSpec((D, D), lambda i: (0, 0)),
            pl.BlockSpec((1, D), lambda i: (0, 0)),
            pl.BlockSpec((D, D), lambda i: (0, 0)),
            pl.BlockSpec((1, D), lambda i: (0, 0)),
        ],
        out_specs=[
            pl.BlockSpec((blk, D), lambda i: (i, 0)),
            pl.BlockSpec((blk, D), lambda i: (i, 0)),
        ],
        out_shape=[
            jax.ShapeDtypeStruct((N, D), jnp.float32),
            jax.ShapeDtypeStruct((N, D), jnp.float32),
        ],
    )(p, bias.reshape(1, D), Wl, bl.reshape(1, D), Wr, br.reshape(1, D))


def _final_body(p_ref, bias_ref, out_ref):
    ps = p_ref[0] + p_ref[1]
    feat = ps[:, :D]
    den = ps[:, D:D + 1]
    out_ref[...] = feat * (1.0 / (den + 1e-16)) + bias_ref[...]


def _final_combine(p, bias):
    grid = (10,)
    blk = N // 10
    return pl.pallas_call(
        _final_body,
        grid=grid,
        in_specs=[
            pl.BlockSpec((2, blk, W_COL), lambda i: (0, i, 0)),
            pl.BlockSpec((1, D), lambda i: (0, 0)),
        ],
        out_specs=pl.BlockSpec((blk, D), lambda i: (i, 0)),
        out_shape=jax.ShapeDtypeStruct((N, D), jnp.float32),
    )(p, bias.reshape(1, D))


def _edge_body(xl_hbm, xr_hbm, src_hbm, dst_hbm, att_hbm, p_hbm,
               src_i, dst_i, a_buf, b_buf, out_buf, att_v,
               accum, sa0, sa1, sb0, sb1, sc0, sc1, si_sem):
    c = lax.axis_index("c")
    s = lax.axis_index("s")
    w = s * NC + c
    base = w * EPW

    pltpu.sync_copy(src_hbm.at[pl.ds(base, SIDX)], src_i.at[0])
    pltpu.sync_copy(dst_hbm.at[pl.ds(base, SIDX)], dst_i.at[0])
    pltpu.sync_copy(att_hbm, att_v)
    att_ks = [att_v[pl.ds(k * 16, 16)] for k in range(8)]
    lane = lax.iota(jnp.int32, 16)
    zero16 = jnp.zeros((16,), jnp.float32)

    def _z(r, carry):
        for k in range(W_COL // 16):
            out_buf[r, pl.ds(k * 16, 16)] = zero16
        return carry
    lax.fori_loop(0, 2 * CH, _z, 0)
    for i in range(ROWS_PER_TILE // (2 * CH)):
        pltpu.sync_copy(out_buf.at[pl.ds(0, 2 * CH)],
                        accum.at[pl.ds(s * ROWS_PER_TILE + i * 2 * CH, 2 * CH)])
    plsc.subcore_barrier()

    sems_a = (sa0, sa1)
    sems_b = (sb0, sb1)
    sems_c = (sc0, sc1)

    def maybe_prefetch(i):
        si = lax.div(i, CPS)
        cond = (lax.rem(i, CPS) == 2) & (si + 1 < NSTAGE)

        def do():
            p = lax.rem(si + 1, 2)
            off = base + (si + 1) * SIDX
            pltpu.async_copy(src_hbm.at[pl.ds(off, SIDX)], src_i.at[p], si_sem)
            pltpu.async_copy(dst_hbm.at[pl.ds(off, SIDX)], dst_i.at[p], si_sem)
        pl.when(cond)(do)

    def maybe_wait_idx(i):
        si = lax.div(i, CPS)
        cond = (lax.rem(i, CPS) == CPS - 3) & (si + 1 < NSTAGE)

        def do():
            pltpu.make_async_copy(src_hbm.at[pl.ds(base, SIDX)],
                                  src_i.at[0], si_sem).wait()
            pltpu.make_async_copy(dst_hbm.at[pl.ds(base, SIDX)],
                                  dst_i.at[0], si_sem).wait()
        pl.when(cond)(do)

    def idx_slice(ref, i):
        p = lax.rem(lax.div(i, CPS), 2)
        o = lax.rem(i, CPS) * CH
        return ref.at[p, pl.ds(o, CH)]

    def issue(i, slot):
        pltpu.async_copy(xl_hbm.at[idx_slice(src_i, i)],
                         a_buf.at[pl.ds(slot * CH, CH)], sems_a[slot])
        pltpu.async_copy(xr_hbm.at[idx_slice(dst_i, i)],
                         b_buf.at[pl.ds(slot * CH, CH)], sems_b[slot])

    def wait(slot):
        pltpu.make_async_copy(xl_hbm.at[src_i.at[0, pl.ds(0, CH)]],
                              a_buf.at[pl.ds(slot * CH, CH)], sems_a[slot]).wait()
        pltpu.make_async_copy(xr_hbm.at[src_i.at[0, pl.ds(0, CH)]],
                              b_buf.at[pl.ds(slot * CH, CH)], sems_b[slot]).wait()

    def wait_scatter(slot):
        pltpu.make_async_copy(out_buf.at[pl.ds(slot * CH, CH)],
                              accum.at[src_i.at[0, pl.ds(0, CH)]],
                              sems_c[slot]).wait()

    def compute_scatter(i, slot):
        rb = slot * CH

        @plsc.parallel_loop(0, CH, 1, unroll=4)
        def _edges(j):
            r = rb + j
            acc = zero16
            avs = []
            for k in range(8):
                a = a_buf[r, pl.ds(k * 16, 16)]
                avs.append(a)
                b = b_buf[r, pl.ds(k * 16, 16)]
                z = a + b
                lrelu = jnp.maximum(z, z * 0.2)
                acc = acc + lrelu * att_ks[k]
            acc2 = acc + lax.rev(acc, (0,))
            parts = [acc2[l] for l in range(8)]
            while len(parts) > 1:
                parts = [parts[m] + parts[m + 1]
                         for m in range(0, len(parts), 2)]
            ex_vec = jnp.exp(parts[0] + zero16)
            for k in range(8):
                out_buf[r, pl.ds(k * 16, 16)] = avs[k] * ex_vec
            out_buf[r, pl.ds(D, 16)] = jnp.where(lane == 0, ex_vec, 0.0)

        pltpu.async_copy(out_buf.at[pl.ds(rb, CH)],
                         accum.at[idx_slice(dst_i, i)],
                         sems_c[slot], add=True)

    issue(0, 0)

    def two_chunks(t, carry):
        i0 = t * 2
        maybe_prefetch(i0)
        maybe_wait_idx(i0)
        issue(i0 + 1, 1)
        wait(0)
        pl.when(t > 0)(lambda: wait_scatter(0))
        compute_scatter(i0, 0)
        maybe_prefetch(i0 + 1)
        maybe_wait_idx(i0 + 1)
        pl.when(i0 + 2 < NCH)(lambda: issue(i0 + 2, 0))
        wait(1)
        pl.when(t > 0)(lambda: wait_scatter(1))
        compute_scatter(i0 + 1, 1)
        return carry

    lax.fori_loop(0, NCH // 2, two_chunks, 0)
    wait_scatter(0)
    wait_scatter(1)

    plsc.subcore_barrier()
    pltpu.sync_copy(accum.at[pl.ds(s * ROWS_PER_TILE, ROWS_PER_TILE)],
                    p_hbm.at[c, pl.ds(s * ROWS_PER_TILE, ROWS_PER_TILE)])


def _edge_pass(xl, xr, src, dst, att):
    mesh = plsc.VectorSubcoreMesh(core_axis_name="c", subcore_axis_name="s")
    return pl.kernel(
        _edge_body,
        out_type=jax.ShapeDtypeStruct((NC, AN, W_COL), jnp.float32),
        mesh=mesh,
        compiler_params=pltpu.CompilerParams(use_tc_tiling_on_sc=False),
        scratch_types=[
            pltpu.VMEM((2, SIDX), jnp.int32),
            pltpu.VMEM((2, SIDX), jnp.int32),
            pltpu.VMEM((2 * CH, D), jnp.float32),
            pltpu.VMEM((2 * CH, D), jnp.float32),
            pltpu.VMEM((2 * CH, W_COL), jnp.float32),
            pltpu.VMEM((D,), jnp.float32),
            pltpu.VMEM_SHARED((AN, W_COL), jnp.float32),
            pltpu.SemaphoreType.DMA,
            pltpu.SemaphoreType.DMA,
            pltpu.SemaphoreType.DMA,
            pltpu.SemaphoreType.DMA,
            pltpu.SemaphoreType.DMA,
            pltpu.SemaphoreType.DMA,
            pltpu.SemaphoreType.DMA,
        ],
    )(xl, xr, src, dst, att)


def kernel(x, edge_idx, W1_l, b1_l, W1_r, b1_r, att1, bias1,
           W2_l, b2_l, W2_r, b2_r, att2, bias2):
    eidx = edge_idx.astype(jnp.int32)
    src = eidx[0]
    dst = eidx[1]
    xl1, xr1 = _linear_pair(x, W1_l, b1_l, W1_r, b1_r)
    p1 = _edge_pass(xl1, xr1, src, dst, att1)
    xl2, xr2 = _combine_linear(p1, bias1, W2_l, b2_l, W2_r, b2_r)
    p2 = _edge_pass(xl2, xr2, src, dst, att2)
    return _final_combine(p2, bias2)


# speedup vs baseline: 1.1107x; 1.0067x over previous
import functools

import jax
import jax.numpy as jnp
from jax import lax
from jax.experimental import pallas as pl
from jax.experimental.pallas import tpu as pltpu
from jax.experimental.pallas import tpu_sc as plsc

N = 10000
E = 320000
D = 128
W_COL = 144
NC = 2
NS = 16
NW = NC * NS
EPW = E // NW
CH = 40
NCH = EPW // CH
CPS = 25
SIDX = CPS * CH
NSTAGE = NCH // CPS
AN = 10240
ROWS_PER_TILE = AN // NS


def _lin_body(x_ref, wl_ref, bl_ref, wr_ref, br_ref, xl_ref, xr_ref):
    xv = x_ref[...]
    xl_ref[...] = jnp.dot(xv, wl_ref[...], preferred_element_type=jnp.float32) + bl_ref[...]
    xr_ref[...] = jnp.dot(xv, wr_ref[...], preferred_element_type=jnp.float32) + br_ref[...]


def _linear_pair(x, Wl, bl, Wr, br):
    grid = (10,)
    blk = N // 10
    return pl.pallas_call(
        _lin_body,
        grid=grid,
        in_specs=[
            pl.BlockSpec((blk, D), lambda i: (i, 0)),
            pl.BlockSpec((D, D), lambda i: (0, 0)),
            pl.BlockSpec((1, D), lambda i: (0, 0)),
            pl.BlockSpec((D, D), lambda i: (0, 0)),
            pl.BlockSpec((1, D), lambda i: (0, 0)),
        ],
        out_specs=[
            pl.BlockSpec((blk, D), lambda i: (i, 0)),
            pl.BlockSpec((blk, D), lambda i: (i, 0)),
        ],
        out_shape=[
            jax.ShapeDtypeStruct((N, D), jnp.float32),
            jax.ShapeDtypeStruct((N, D), jnp.float32),
        ],
    )(x, Wl, bl.reshape(1, D), Wr, br.reshape(1, D))


def _combine_lin_body(p_ref, bias_ref, wl_ref, bl_ref, wr_ref, br_ref, xl_ref, xr_ref):
    ps = p_ref[0] + p_ref[1]
    feat = ps[:, :D]
    den = ps[:, D:D + 1]
    h = jnp.maximum(feat * (1.0 / (den + 1e-16)) + bias_ref[...], 0.0)
    xl_ref[...] = jnp.dot(h, wl_ref[...], preferred_element_type=jnp.float32) + bl_ref[...]
    xr_ref[...] = jnp.dot(h, wr_ref[...], preferred_element_type=jnp.float32) + br_ref[...]


def _combine_linear(p, bias, Wl, bl, Wr, br):
    grid = (10,)
    blk = N // 10
    return pl.pallas_call(
        _combine_lin_body,
        grid=grid,
        in_specs=[
            pl.BlockSpec((2, blk, W_COL), lambda i: (0, i, 0)),
            pl.BlockSpec((1, D), lambda i: (0, 0)),
            pl.BlockSpec((D, D), lambda i: (0, 0)),
            pl.BlockSpec((1, D), lambda i: (0, 0)),
            pl.BlockSpec((D, D), lambda i: (0, 0)),
            pl.BlockSpec((1, D), lambda i: (0, 0)),
        ],
        out_specs=[
            pl.BlockSpec((blk, D), lambda i: (i, 0)),
            pl.BlockSpec((blk, D), lambda i: (i, 0)),
        ],
        out_shape=[
            jax.ShapeDtypeStruct((N, D), jnp.float32),
            jax.ShapeDtypeStruct((N, D), jnp.float32),
        ],
    )(p, bias.reshape(1, D), Wl, bl.reshape(1, D), Wr, br.reshape(1, D))


def _final_body(p_ref, bias_ref, out_ref):
    ps = p_ref[0] + p_ref[1]
    feat = ps[:, :D]
    den = ps[:, D:D + 1]
    out_ref[...] = feat * (1.0 / (den + 1e-16)) + bias_ref[...]


def _final_combine(p, bias):
    grid = (10,)
    blk = N // 10
    return pl.pallas_call(
        _final_body,
        grid=grid,
        in_specs=[
            pl.BlockSpec((2, blk, W_COL), lambda i: (0, i, 0)),
            pl.BlockSpec((1, D), lambda i: (0, 0)),
        ],
        out_specs=pl.BlockSpec((blk, D), lambda i: (i, 0)),
        out_shape=jax.ShapeDtypeStruct((N, D), jnp.float32),
    )(p, bias.reshape(1, D))


def _edge_body(xl_hbm, xr_hbm, src_hbm, dst_hbm, att_hbm, p_hbm,
               src_i, dst_i, a_buf, b_buf, out_buf, att_v,
               accum, sa0, sa1, sb0, sb1, sc0, sc1, si_sem):
    c = lax.axis_index("c")
    s = lax.axis_index("s")
    w = s * NC + c
    base = w * EPW

    pltpu.sync_copy(src_hbm.at[pl.ds(base, SIDX)], src_i.at[0])
    pltpu.sync_copy(dst_hbm.at[pl.ds(base, SIDX)], dst_i.at[0])
    pltpu.sync_copy(att_hbm, att_v)
    att_ks = [att_v[pl.ds(k * 16, 16)] for k in range(8)]
    lane = lax.iota(jnp.int32, 16)
    zero16 = jnp.zeros((16,), jnp.float32)

    def _z(r, carry):
        for k in range(W_COL // 16):
            out_buf[r, pl.ds(k * 16, 16)] = zero16
        return carry
    lax.fori_loop(0, 2 * CH, _z, 0)
    for i in range(ROWS_PER_TILE // (2 * CH)):
        pltpu.sync_copy(out_buf.at[pl.ds(0, 2 * CH)],
                        accum.at[pl.ds(s * ROWS_PER_TILE + i * 2 * CH, 2 * CH)])
    plsc.subcore_barrier()

    sems_a = (sa0, sa1)
    sems_b = (sb0, sb1)
    sems_c = (sc0, sc1)

    def maybe_prefetch(i):
        si = lax.div(i, CPS)
        cond = (lax.rem(i, CPS) == 2) & (si + 1 < NSTAGE)

        def do():
            p = lax.rem(si + 1, 2)
            off = base + (si + 1) * SIDX
            pltpu.async_copy(src_hbm.at[pl.ds(off, SIDX)], src_i.at[p], si_sem)
            pltpu.async_copy(dst_hbm.at[pl.ds(off, SIDX)], dst_i.at[p], si_sem)
        pl.when(cond)(do)

    def maybe_wait_idx(i):
        si = lax.div(i, CPS)
        cond = (lax.rem(i, CPS) == CPS - 3) & (si + 1 < NSTAGE)

        def do():
            pltpu.make_async_copy(src_hbm.at[pl.ds(base, SIDX)],
                                  src_i.at[0], si_sem).wait()
            pltpu.make_async_copy(dst_hbm.at[pl.ds(base, SIDX)],
                                  dst_i.at[0], si_sem).wait()
        pl.when(cond)(do)

    def idx_slice(ref, i):
        p = lax.rem(lax.div(i, CPS), 2)
        o = lax.rem(i, CPS) * CH
        return ref.at[p, pl.ds(o, CH)]

    def issue(i, slot):
        pltpu.async_copy(xl_hbm.at[idx_slice(src_i, i)],
                         a_buf.at[pl.ds(slot * CH, CH)], sems_a[slot])
        pltpu.async_copy(xr_hbm.at[idx_slice(dst_i, i)],
                         b_buf.at[pl.ds(slot * CH, CH)], sems_b[slot])

    def wait(slot):
        pltpu.make_async_copy(xl_hbm.at[src_i.at[0, pl.ds(0, CH)]],
                              a_buf.at[pl.ds(slot * CH, CH)], sems_a[slot]).wait()
        pltpu.make_async_copy(xr_hbm.at[src_i.at[0, pl.ds(0, CH)]],
                              b_buf.at[pl.ds(slot * CH, CH)], sems_b[slot]).wait()

    def wait_scatter(slot):
        pltpu.make_async_copy(out_buf.at[pl.ds(slot * CH, CH)],
                              accum.at[src_i.at[0, pl.ds(0, CH)]],
                              sems_c[slot]).wait()

    def compute_scatter(i, slot):
        rb = slot * CH

        @plsc.parallel_loop(0, CH, 1, unroll=8)
        def _edges(j):
            r = rb + j
            acc = zero16
            avs = []
            for k in range(8):
                a = a_buf[r, pl.ds(k * 16, 16)]
                avs.append(a)
                b = b_buf[r, pl.ds(k * 16, 16)]
                z = a + b
                lrelu = jnp.maximum(z, z * 0.2)
                acc = acc + lrelu * att_ks[k]
            acc2 = acc + lax.rev(acc, (0,))
            parts = [acc2[l] for l in range(8)]
            while len(parts) > 1:
                parts = [parts[m] + parts[m + 1]
                         for m in range(0, len(parts), 2)]
            ex_vec = jnp.exp(parts[0] + zero16)
            for k in range(8):
                out_buf[r, pl.ds(k * 16, 16)] = avs[k] * ex_vec
            out_buf[r, pl.ds(D, 16)] = jnp.where(lane == 0, ex_vec, 0.0)

        pltpu.async_copy(out_buf.at[pl.ds(rb, CH)],
                         accum.at[idx_slice(dst_i, i)],
                         sems_c[slot], add=True)

    issue(0, 0)

    def two_chunks(t, carry):
        i0 = t * 2
        maybe_prefetch(i0)
        maybe_wait_idx(i0)
        issue(i0 + 1, 1)
        wait(0)
        pl.when(t > 0)(lambda: wait_scatter(0))
        compute_scatter(i0, 0)
        maybe_prefetch(i0 + 1)
        maybe_wait_idx(i0 + 1)
        pl.when(i0 + 2 < NCH)(lambda: issue(i0 + 2, 0))
        wait(1)
        pl.when(t > 0)(lambda: wait_scatter(1))
        compute_scatter(i0 + 1, 1)
        return carry

    lax.fori_loop(0, NCH // 2, two_chunks, 0)
    wait_scatter(0)
    wait_scatter(1)

    plsc.subcore_barrier()
    pltpu.sync_copy(accum.at[pl.ds(s * ROWS_PER_TILE, ROWS_PER_TILE)],
                    p_hbm.at[c, pl.ds(s * ROWS_PER_TILE, ROWS_PER_TILE)])


def _edge_pass(xl, xr, src, dst, att):
    mesh = plsc.VectorSubcoreMesh(core_axis_name="c", subcore_axis_name="s")
    return pl.kernel(
        _edge_body,
        out_type=jax.ShapeDtypeStruct((NC, AN, W_COL), jnp.float32),
        mesh=mesh,
        compiler_params=pltpu.CompilerParams(use_tc_tiling_on_sc=False),
        scratch_types=[
            pltpu.VMEM((2, SIDX), jnp.int32),
            pltpu.VMEM((2, SIDX), jnp.int32),
            pltpu.VMEM((2 * CH, D), jnp.float32),
            pltpu.VMEM((2 * CH, D), jnp.float32),
            pltpu.VMEM((2 * CH, W_COL), jnp.float32),
            pltpu.VMEM((D,), jnp.float32),
            pltpu.VMEM_SHARED((AN, W_COL), jnp.float32),
            pltpu.SemaphoreType.DMA,
            pltpu.SemaphoreType.DMA,
            pltpu.SemaphoreType.DMA,
            pltpu.SemaphoreType.DMA,
            pltpu.SemaphoreType.DMA,
            pltpu.SemaphoreType.DMA,
            pltpu.SemaphoreType.DMA,
        ],
    )(xl, xr, src, dst, att)


def kernel(x, edge_idx, W1_l, b1_l, W1_r, b1_r, att1, bias1,
           W2_l, b2_l, W2_r, b2_r, att2, bias2):
    eidx = edge_idx.astype(jnp.int32)
    src = eidx[0]
    dst = eidx[1]
    xl1, xr1 = _linear_pair(x, W1_l, b1_l, W1_r, b1_r)
    p1 = _edge_pass(xl1, xr1, src, dst, att1)
    xl2, xr2 = _combine_linear(p1, bias1, W2_l, b2_l, W2_r, b2_r)
    p2 = _edge_pass(xl2, xr2, src, dst, att2)
    return _final_combine(p2, bias2)
